# Initial kernel scaffold; baseline (speedup 1.0000x reference)
import functools

import jax
import jax.numpy as jnp
from jax import lax
from jax.experimental import pallas as pl
from jax.experimental.pallas import tpu as pltpu
from jax.experimental.pallas import tpu_sc as plsc

NC = 2
NS = 16
L = 16
W = NC * NS

P = 2_000_000
PW = P // W
CP = 8192
NFULL = PW // CP
TAILP = PW - NFULL * CP
TAIL_FULL = TAILP // L
TAIL_REM = TAILP - TAIL_FULL * L
CE = 2 * CP
TE = 2 * TAILP
MARGIN = 1.0
INV_P = 1.0 / P


@functools.partial(
    pl.kernel,
    out_type=jax.ShapeDtypeStruct((W, L), jnp.float32),
    mesh=plsc.VectorSubcoreMesh(
        core_axis_name="c", subcore_axis_name="s",
        num_cores=NC, num_subcores=NS),
    scratch_types=[
        pltpu.VMEM((CE,), jnp.int32),
        pltpu.VMEM((CE,), jnp.float32),
        pltpu.VMEM((TE,), jnp.int32),
        pltpu.VMEM((TE,), jnp.float32),
        pltpu.VMEM((L,), jnp.float32),
        pltpu.SemaphoreType.DMA,
    ],
)
def _sc_loss(outputs_hbm, idx_hbm, out_hbm, idx_v, val_v, idx_t, val_t,
             acc_v, sem):
    wid = lax.axis_index("s") * NC + lax.axis_index("c")
    base = wid * 2 * PW
    iota = lax.iota(jnp.int32, L)
    ev = iota * 2
    od = ev + 1
    tail_mask = iota < TAIL_REM

    def accum_pairs(val_ref, n_vregs, acc):
        def it(t, a):
            b = t * (2 * L)
            x0 = plsc.load_gather(val_ref, [b + ev])
            x1 = plsc.load_gather(val_ref, [b + od])
            return a + jnp.maximum(MARGIN - x0 + x1, 0.0)
        return lax.fori_loop(0, n_vregs, it, acc)

    def chunk(c, acc):
        start = pl.multiple_of(base + c * CE, 8)
        pltpu.sync_copy(idx_hbm.at[pl.ds(start, CE)], idx_v)
        pltpu.async_copy(outputs_hbm.at[idx_v], val_v, sem).wait()
        return accum_pairs(val_v, CP // L, acc)

    acc = lax.fori_loop(0, NFULL, chunk, jnp.zeros((L,), jnp.float32))

    tstart = pl.multiple_of(base + NFULL * CE, 8)
    pltpu.sync_copy(idx_hbm.at[pl.ds(tstart, TE)], idx_t)
    pltpu.async_copy(outputs_hbm.at[idx_t], val_t, sem).wait()
    acc = accum_pairs(val_t, TAIL_FULL, acc)
    b = TAIL_FULL * (2 * L)
    x0 = plsc.load_gather(val_t, [jnp.minimum(b + ev, TE - 1)])
    x1 = plsc.load_gather(val_t, [jnp.minimum(b + od, TE - 1)])
    u = jnp.maximum(MARGIN - x0 + x1, 0.0)
    acc = acc + jnp.where(tail_mask, u, 0.0)

    acc_v[...] = acc * INV_P
    pltpu.sync_copy(acc_v, out_hbm.at[wid])


def kernel(outputs, mask):
    idx_flat = mask.astype(jnp.int32).reshape(-1)
    parts = _sc_loss(outputs, idx_flat)
    return jnp.sum(parts)


# speedup vs baseline: 10.3763x; 10.3763x over previous
import functools

import jax
import jax.numpy as jnp
from jax import lax
from jax.experimental import pallas as pl
from jax.experimental.pallas import tpu as pltpu
from jax.experimental.pallas import tpu_sc as plsc

NC = 2
NS = 16
L = 16
W = NC * NS

P = 2_000_000
PW = P // W
CP = 8192
NFULL = PW // CP
TAILP = PW - NFULL * CP
TAIL_FULL = TAILP // L
TAIL_REM = TAILP - TAIL_FULL * L
CE = 2 * CP
TE = 2 * TAILP
MARGIN = 1.0
INV_P = 1.0 / P


@functools.partial(
    pl.kernel,
    out_type=jax.ShapeDtypeStruct((W, L), jnp.float32),
    mesh=plsc.VectorSubcoreMesh(
        core_axis_name="c", subcore_axis_name="s",
        num_cores=NC, num_subcores=NS),
    scratch_types=[
        pltpu.VMEM((CE,), jnp.int32),
        pltpu.VMEM((CE,), jnp.float32),
        pltpu.VMEM((TE,), jnp.int32),
        pltpu.VMEM((TE,), jnp.float32),
        pltpu.VMEM((L,), jnp.float32),
        pltpu.SemaphoreType.DMA,
    ],
    compiler_params=pltpu.CompilerParams(needs_layout_passes=False),
)
def _sc_loss(outputs_hbm, idx_hbm, out_hbm, idx_v, val_v, idx_t, val_t,
             acc_v, sem):
    wid = lax.axis_index("s") * NC + lax.axis_index("c")
    base = wid * 2 * PW
    iota = lax.iota(jnp.int32, L)
    ev = iota * 2
    od = ev + 1
    tail_mask = iota < TAIL_REM

    def accum_pairs(val_ref, n_vregs, acc):
        def it(t, a):
            b = t * (2 * L)
            x0 = plsc.load_gather(val_ref, [b + ev])
            x1 = plsc.load_gather(val_ref, [b + od])
            return a + jnp.maximum(MARGIN - x0 + x1, 0.0)
        return lax.fori_loop(0, n_vregs, it, acc)

    def chunk(c, acc):
        start = pl.multiple_of(base + c * CE, 8)
        pltpu.sync_copy(idx_hbm.at[pl.ds(start, CE)], idx_v)
        pltpu.async_copy(outputs_hbm.at[idx_v], val_v, sem).wait()
        return accum_pairs(val_v, CP // L, acc)

    acc = lax.fori_loop(0, NFULL, chunk, jnp.zeros((L,), jnp.float32))

    tstart = pl.multiple_of(base + NFULL * CE, 8)
    pltpu.sync_copy(idx_hbm.at[pl.ds(tstart, TE)], idx_t)
    pltpu.async_copy(outputs_hbm.at[idx_t], val_t, sem).wait()
    acc = accum_pairs(val_t, TAIL_FULL, acc)
    b = TAIL_FULL * (2 * L)
    x0 = plsc.load_gather(val_t, [jnp.minimum(b + ev, TE - 1)])
    x1 = plsc.load_gather(val_t, [jnp.minimum(b + od, TE - 1)])
    u = jnp.maximum(MARGIN - x0 + x1, 0.0)
    acc = acc + jnp.where(tail_mask, u, 0.0)

    acc_v[...] = acc * INV_P
    pltpu.sync_copy(acc_v, out_hbm.at[wid])


def kernel(outputs, mask):
    idx_flat = mask.astype(jnp.int32).reshape(-1)
    parts = _sc_loss(outputs, idx_flat)
    return jnp.sum(parts)


# speedup vs baseline: 101.9082x; 9.8213x over previous
import functools

import jax
import jax.numpy as jnp
from jax import lax
from jax.experimental import pallas as pl
from jax.experimental.pallas import tpu as pltpu
from jax.experimental.pallas import tpu_sc as plsc

NC = 2
NS = 16
L = 16
W = NC * NS

P = 2_000_000
PW = 62496
CP = 8192
NCHUNK = PW // CP
TAILP = PW - NCHUNK * CP
XBASE = W * PW
NX = (P - XBASE) // L
MARGIN = 1.0
INV_P = 1.0 / P


@functools.partial(
    pl.kernel,
    out_type=jax.ShapeDtypeStruct((W, L), jnp.float32),
    mesh=plsc.VectorSubcoreMesh(
        core_axis_name="c", subcore_axis_name="s",
        num_cores=NC, num_subcores=NS),
    scratch_types=[
        pltpu.VMEM((CP,), jnp.int32),
        pltpu.VMEM((CP,), jnp.int32),
        pltpu.VMEM((CP,), jnp.float32),
        pltpu.VMEM((CP,), jnp.float32),
        pltpu.VMEM((L,), jnp.int32),
        pltpu.VMEM((L,), jnp.int32),
        pltpu.VMEM((L,), jnp.float32),
        pltpu.VMEM((L,), jnp.float32),
        pltpu.VMEM((L,), jnp.float32),
        pltpu.SemaphoreType.DMA,
    ],
)
def _sc_loss(outputs_hbm, idxl_hbm, idxr_hbm, out_hbm,
             idxl_v, idxr_v, vall_v, valr_v,
             idxl_x, idxr_x, vall_x, valr_x, acc_v, sem):
    wid = lax.axis_index("s") * NC + lax.axis_index("c")
    base = wid * PW

    def accum(val0_ref, val1_ref, n_vregs, acc):
        def it(t, a):
            x0 = val0_ref[pl.ds(t * L, L)]
            x1 = val1_ref[pl.ds(t * L, L)]
            return a + jnp.maximum(MARGIN - x0 + x1, 0.0)
        return lax.fori_loop(0, n_vregs, it, acc)

    def run_chunk(start, n_pairs, acc):
        pltpu.sync_copy(idxl_hbm.at[pl.ds(start, n_pairs)],
                        idxl_v.at[pl.ds(0, n_pairs)])
        pltpu.sync_copy(idxr_hbm.at[pl.ds(start, n_pairs)],
                        idxr_v.at[pl.ds(0, n_pairs)])
        cl = pltpu.async_copy(
            outputs_hbm.at[idxl_v.at[pl.ds(0, n_pairs)]],
            vall_v.at[pl.ds(0, n_pairs)], sem)
        cr = pltpu.async_copy(
            outputs_hbm.at[idxr_v.at[pl.ds(0, n_pairs)]],
            valr_v.at[pl.ds(0, n_pairs)], sem)
        cl.wait()
        cr.wait()
        return accum(vall_v, valr_v, n_pairs // L, acc)

    def chunk(c, acc):
        return run_chunk(pl.multiple_of(base + c * CP, 8), CP, acc)

    acc = lax.fori_loop(0, NCHUNK, chunk, jnp.zeros((L,), jnp.float32))
    acc = run_chunk(pl.multiple_of(base + NCHUNK * CP, 8), TAILP, acc)

    @pl.when(wid < NX)
    def _():
        xs = pl.multiple_of(XBASE + wid * L, 8)
        pltpu.sync_copy(idxl_hbm.at[pl.ds(xs, L)], idxl_x)
        pltpu.sync_copy(idxr_hbm.at[pl.ds(xs, L)], idxr_x)
        cl = pltpu.async_copy(outputs_hbm.at[idxl_x], vall_x, sem)
        cr = pltpu.async_copy(outputs_hbm.at[idxr_x], valr_x, sem)
        cl.wait()
        cr.wait()
        x0 = vall_x[...]
        x1 = valr_x[...]
        acc_v[...] = (acc + jnp.maximum(MARGIN - x0 + x1, 0.0)) * INV_P

    @pl.when(wid >= NX)
    def _():
        acc_v[...] = acc * INV_P

    pltpu.sync_copy(acc_v, out_hbm.at[wid])


def kernel(outputs, mask):
    m = mask.astype(jnp.int32)
    parts = _sc_loss(outputs, m[:, 0], m[:, 1])
    return jnp.sum(parts)


# speedup vs baseline: 108.3579x; 1.0633x over previous
import functools

import jax
import jax.numpy as jnp
from jax import lax
from jax.experimental import pallas as pl
from jax.experimental.pallas import tpu as pltpu
from jax.experimental.pallas import tpu_sc as plsc

NC = 2
NS = 16
L = 16
W = NC * NS

P = 2_000_000
PW = 62496
CP = 10416
NCHUNK = PW // CP
NV = CP // L
XBASE = W * PW
NX = (P - XBASE) // L
MARGIN = 1.0
INV_P = 1.0 / P


@functools.partial(
    pl.kernel,
    out_type=jax.ShapeDtypeStruct((W, L), jnp.float32),
    mesh=plsc.VectorSubcoreMesh(
        core_axis_name="c", subcore_axis_name="s",
        num_cores=NC, num_subcores=NS),
    scratch_types=[
        pltpu.VMEM((CP,), jnp.int32),
        pltpu.VMEM((CP,), jnp.int32),
        pltpu.VMEM((CP,), jnp.int32),
        pltpu.VMEM((CP,), jnp.int32),
        pltpu.VMEM((CP,), jnp.float32),
        pltpu.VMEM((CP,), jnp.float32),
        pltpu.VMEM((CP,), jnp.float32),
        pltpu.VMEM((CP,), jnp.float32),
        pltpu.VMEM((L,), jnp.int32),
        pltpu.VMEM((L,), jnp.int32),
        pltpu.VMEM((L,), jnp.float32),
        pltpu.VMEM((L,), jnp.float32),
        pltpu.VMEM((L,), jnp.float32),
        [pltpu.SemaphoreType.DMA] * 2,
        [pltpu.SemaphoreType.DMA] * 2,
    ],
)
def _sc_loss(outputs_hbm, idxl_hbm, idxr_hbm, out_hbm,
             idxl0, idxl1, idxr0, idxr1, vall0, vall1, valr0, valr1,
             idxl_x, idxr_x, vall_x, valr_x, acc_v, isems, gsems):
    wid = lax.axis_index("s") * NC + lax.axis_index("c")
    base = wid * PW
    idxl = [idxl0, idxl1]
    idxr = [idxr0, idxr1]
    vall = [vall0, vall1]
    valr = [valr0, valr1]

    def start_idx(c, b):
        start = pl.multiple_of(base + c * CP, 8)
        pltpu.async_copy(idxl_hbm.at[pl.ds(start, CP)], idxl[b], isems[b])
        pltpu.async_copy(idxr_hbm.at[pl.ds(start, CP)], idxr[b], isems[b])

    def wait_idx(b):
        pltpu.make_async_copy(idxl_hbm.at[pl.ds(0, CP)], idxl[b],
                              isems[b]).wait()
        pltpu.make_async_copy(idxr_hbm.at[pl.ds(0, CP)], idxr[b],
                              isems[b]).wait()

    def start_gather(b):
        pltpu.async_copy(outputs_hbm.at[idxl[b]], vall[b], gsems[b])
        pltpu.async_copy(outputs_hbm.at[idxr[b]], valr[b], gsems[b])

    def wait_gather(b):
        pltpu.make_async_copy(outputs_hbm.at[idxl[b]], vall[b],
                              gsems[b]).wait()
        pltpu.make_async_copy(outputs_hbm.at[idxr[b]], valr[b],
                              gsems[b]).wait()

    def accum(b, acc):
        def it(t, a):
            x0 = vall[b][pl.ds(t * L, L)]
            x1 = valr[b][pl.ds(t * L, L)]
            return a + jnp.maximum(MARGIN - x0 + x1, 0.0)
        return lax.fori_loop(0, NV, it, acc)

    start_idx(0, 0)
    start_idx(1, 1)
    wait_idx(0)
    start_gather(0)

    def two_chunks(o, acc):
        c = o * 2
        wait_idx(1)
        start_gather(1)
        wait_gather(0)
        acc = accum(0, acc)
        @pl.when(c + 2 < NCHUNK)
        def _():
            start_idx(c + 2, 0)
            wait_idx(0)
            start_gather(0)
        wait_gather(1)
        acc = accum(1, acc)
        @pl.when(c + 3 < NCHUNK)
        def _():
            start_idx(c + 3, 1)
        return acc

    acc = lax.fori_loop(0, NCHUNK // 2, two_chunks,
                        jnp.zeros((L,), jnp.float32))

    @pl.when(wid < NX)
    def _():
        xs = pl.multiple_of(XBASE + wid * L, 8)
        pltpu.sync_copy(idxl_hbm.at[pl.ds(xs, L)], idxl_x)
        pltpu.sync_copy(idxr_hbm.at[pl.ds(xs, L)], idxr_x)
        cl = pltpu.async_copy(outputs_hbm.at[idxl_x], vall_x, gsems[0])
        cr = pltpu.async_copy(outputs_hbm.at[idxr_x], valr_x, gsems[0])
        cl.wait()
        cr.wait()
        x0 = vall_x[...]
        x1 = valr_x[...]
        acc_v[...] = (acc + jnp.maximum(MARGIN - x0 + x1, 0.0)) * INV_P

    @pl.when(wid >= NX)
    def _():
        acc_v[...] = acc * INV_P

    pltpu.sync_copy(acc_v, out_hbm.at[wid])


def kernel(outputs, mask):
    m = mask.astype(jnp.int32)
    parts = _sc_loss(outputs, m[:, 0], m[:, 1])
    return jnp.sum(parts)


# speedup vs baseline: 178.1530x; 1.6441x over previous
import functools

import jax
import jax.numpy as jnp
from jax import lax
from jax.experimental import pallas as pl
from jax.experimental.pallas import tpu as pltpu
from jax.experimental.pallas import tpu_sc as plsc

NC = 2
NS = 16
L = 16
W = NC * NS

P = 2_000_000
PW = 62496
CP = 4464
NCHUNK = PW // CP
NV = CP // L
XBASE = W * PW
NX = (P - XBASE) // L
V = 1_000_000
VSTG = 62496
VSTG_LAST = V - (NS - 1) * VSTG
MARGIN = 1.0
INV_P = 1.0 / P


@functools.partial(
    pl.kernel,
    out_type=jax.ShapeDtypeStruct((W, L), jnp.float32),
    mesh=plsc.VectorSubcoreMesh(
        core_axis_name="c", subcore_axis_name="s",
        num_cores=NC, num_subcores=NS),
    scratch_types=[
        pltpu.VMEM((CP,), jnp.int32),
        pltpu.VMEM((CP,), jnp.int32),
        pltpu.VMEM((CP,), jnp.int32),
        pltpu.VMEM((CP,), jnp.int32),
        pltpu.VMEM((CP,), jnp.float32),
        pltpu.VMEM((CP,), jnp.float32),
        pltpu.VMEM((CP,), jnp.float32),
        pltpu.VMEM((CP,), jnp.float32),
        pltpu.VMEM((L,), jnp.int32),
        pltpu.VMEM((L,), jnp.int32),
        pltpu.VMEM((L,), jnp.float32),
        pltpu.VMEM((L,), jnp.float32),
        pltpu.VMEM((L,), jnp.float32),
        pltpu.VMEM_SHARED((V,), jnp.float32),
        [pltpu.SemaphoreType.DMA] * 2,
        [pltpu.SemaphoreType.DMA] * 2,
        pltpu.SemaphoreType.DMA,
    ],
)
def _sc_loss(outputs_hbm, idxl_hbm, idxr_hbm, out_hbm,
             idxl0, idxl1, idxr0, idxr1, vall0, vall1, valr0, valr1,
             idxl_x, idxr_x, vall_x, valr_x, acc_v, table_sh,
             isems, gsems, ssem):
    sid = lax.axis_index("s")
    wid = sid * NC + lax.axis_index("c")
    base = wid * PW
    idxl = [idxl0, idxl1]
    idxr = [idxr0, idxr1]
    vall = [vall0, vall1]
    valr = [valr0, valr1]

    def start_idx(c, b):
        start = pl.multiple_of(base + c * CP, 8)
        pltpu.async_copy(idxl_hbm.at[pl.ds(start, CP)], idxl[b], isems[b])
        pltpu.async_copy(idxr_hbm.at[pl.ds(start, CP)], idxr[b], isems[b])

    def wait_idx(b):
        pltpu.make_async_copy(idxl_hbm.at[pl.ds(0, CP)], idxl[b],
                              isems[b]).wait()
        pltpu.make_async_copy(idxr_hbm.at[pl.ds(0, CP)], idxr[b],
                              isems[b]).wait()

    def start_gather(b):
        pltpu.async_copy(table_sh.at[idxl[b]], vall[b], gsems[b])
        pltpu.async_copy(table_sh.at[idxr[b]], valr[b], gsems[b])

    def wait_gather(b):
        pltpu.make_async_copy(table_sh.at[idxl[b]], vall[b],
                              gsems[b]).wait()
        pltpu.make_async_copy(table_sh.at[idxr[b]], valr[b],
                              gsems[b]).wait()

    def accum(b, acc):
        def it(t, a):
            x0 = vall[b][pl.ds(t * L, L)]
            x1 = valr[b][pl.ds(t * L, L)]
            return a + jnp.maximum(MARGIN - x0 + x1, 0.0)
        return lax.fori_loop(0, NV, it, acc)

    start_idx(0, 0)
    start_idx(1, 1)

    def stage_piece(p, _):
        poff = pl.multiple_of(sid * VSTG + p * CP, 8)
        pltpu.async_copy(outputs_hbm.at[pl.ds(poff, CP)], vall0, ssem).wait()
        pltpu.async_copy(vall0, table_sh.at[pl.ds(poff, CP)], ssem).wait()
        return 0

    lax.fori_loop(0, VSTG // CP, stage_piece, 0)

    @pl.when(sid == NS - 1)
    def _():
        poff = pl.multiple_of(NS * VSTG, 8)
        rem = V - NS * VSTG
        pltpu.async_copy(outputs_hbm.at[pl.ds(poff, rem)],
                         vall1.at[pl.ds(0, rem)], ssem).wait()
        pltpu.async_copy(vall1.at[pl.ds(0, rem)],
                         table_sh.at[pl.ds(poff, rem)], ssem).wait()

    plsc.subcore_barrier()
    wait_idx(0)
    start_gather(0)

    def two_chunks(o, acc):
        c = o * 2
        wait_idx(1)
        start_gather(1)
        wait_gather(0)
        acc = accum(0, acc)
        @pl.when(c + 2 < NCHUNK)
        def _():
            start_idx(c + 2, 0)
            wait_idx(0)
            start_gather(0)
        wait_gather(1)
        acc = accum(1, acc)
        @pl.when(c + 3 < NCHUNK)
        def _():
            start_idx(c + 3, 1)
        return acc

    acc = lax.fori_loop(0, NCHUNK // 2, two_chunks,
                        jnp.zeros((L,), jnp.float32))

    @pl.when(wid < NX)
    def _():
        xs = pl.multiple_of(XBASE + wid * L, 8)
        pltpu.sync_copy(idxl_hbm.at[pl.ds(xs, L)], idxl_x)
        pltpu.sync_copy(idxr_hbm.at[pl.ds(xs, L)], idxr_x)
        cl = pltpu.async_copy(table_sh.at[idxl_x], vall_x, gsems[0])
        cr = pltpu.async_copy(table_sh.at[idxr_x], valr_x, gsems[0])
        cl.wait()
        cr.wait()
        x0 = vall_x[...]
        x1 = valr_x[...]
        acc_v[...] = (acc + jnp.maximum(MARGIN - x0 + x1, 0.0)) * INV_P

    @pl.when(wid >= NX)
    def _():
        acc_v[...] = acc * INV_P

    pltpu.sync_copy(acc_v, out_hbm.at[wid])


def kernel(outputs, mask):
    m = mask.astype(jnp.int32)
    parts = _sc_loss(outputs, m[:, 0], m[:, 1])
    return jnp.sum(parts)


# speedup vs baseline: 352.3417x; 1.9777x over previous
import functools

import jax
import jax.numpy as jnp
from jax import lax
from jax.experimental import pallas as pl
from jax.experimental.pallas import tpu as pltpu
from jax.experimental.pallas import tpu_sc as plsc

NC = 2
NS = 16
L = 16
W = NC * NS

P = 2_000_000
HB = 128
NB = P // HB
BW = 488
CB = 61
NCHUNK = BW // CB
NXB = NB - W * BW
CPW = CB * 2 * HB
V = 1_000_000
VSTG = 62496
SPIECE = 3472
MARGIN = 1.0
INV_P = 1.0 / P


@functools.partial(
    pl.kernel,
    out_type=jax.ShapeDtypeStruct((W, L), jnp.float32),
    mesh=plsc.VectorSubcoreMesh(
        core_axis_name="c", subcore_axis_name="s",
        num_cores=NC, num_subcores=NS),
    scratch_types=[
        pltpu.VMEM((CB, 2, HB), jnp.int32),
        pltpu.VMEM((CPW,), jnp.int32),
        pltpu.VMEM((CPW,), jnp.float32),
        pltpu.VMEM((CPW,), jnp.float32),
        pltpu.VMEM((SPIECE,), jnp.float32),
        pltpu.VMEM((L,), jnp.float32),
        pltpu.VMEM_SHARED((V,), jnp.float32),
        pltpu.SemaphoreType.DMA,
        [pltpu.SemaphoreType.DMA] * 2,
        pltpu.SemaphoreType.DMA,
    ],
)
def _sc_loss(idx3_hbm, outputs_hbm, out_hbm,
             idx3_v, idx1d, val_v0, val_v1, stg_v, acc_v, table_sh,
             isem, gsems, ssem):
    sid = lax.axis_index("s")
    wid = sid * NC + lax.axis_index("c")
    base = wid * BW
    val = [val_v0, val_v1]

    def start_idx(c):
        pltpu.async_copy(idx3_hbm.at[pl.ds(base + c * CB, CB)], idx3_v, isem)

    def wait_idx():
        pltpu.make_async_copy(idx3_hbm.at[pl.ds(0, CB)], idx3_v, isem).wait()

    def deinterleave():
        def row(t, _):
            b = t * (2 * HB)
            for k in range(HB // L):
                idx1d[pl.ds(b + k * L, L)] = idx3_v[t, 0, pl.ds(k * L, L)]
                idx1d[pl.ds(b + HB + k * L, L)] = idx3_v[t, 1, pl.ds(k * L, L)]
            return 0
        lax.fori_loop(0, CB, row, 0)

    def start_gather(b):
        pltpu.async_copy(table_sh.at[idx1d], val[b], gsems[b])

    def wait_gather(b):
        pltpu.make_async_copy(table_sh.at[idx1d], val[b], gsems[b]).wait()

    def accum(b, acc):
        def it(t, a):
            bb = t * (2 * HB)
            for k in range(HB // L):
                x0 = val[b][pl.ds(bb + k * L, L)]
                x1 = val[b][pl.ds(bb + HB + k * L, L)]
                a = a + jnp.maximum(MARGIN - x0 + x1, 0.0)
            return a
        return lax.fori_loop(0, CB, it, acc)

    start_idx(0)

    def stage_piece(p, _):
        poff = pl.multiple_of(sid * VSTG + p * SPIECE, 8)
        pltpu.async_copy(outputs_hbm.at[pl.ds(poff, SPIECE)], stg_v,
                         ssem).wait()
        pltpu.async_copy(stg_v, table_sh.at[pl.ds(poff, SPIECE)],
                         ssem).wait()
        return 0

    lax.fori_loop(0, VSTG // SPIECE, stage_piece, 0)

    @pl.when(sid == NS - 1)
    def _():
        poff = pl.multiple_of(NS * VSTG, 8)
        rem = V - NS * VSTG
        pltpu.async_copy(outputs_hbm.at[pl.ds(poff, rem)],
                         stg_v.at[pl.ds(0, rem)], ssem).wait()
        pltpu.async_copy(stg_v.at[pl.ds(0, rem)],
                         table_sh.at[pl.ds(poff, rem)], ssem).wait()

    plsc.subcore_barrier()

    acc = jnp.zeros((L,), jnp.float32)
    for c in range(NCHUNK):
        wait_idx()
        if c > 0:
            wait_gather((c - 1) % 2)
        deinterleave()
        if c < NCHUNK - 1:
            start_idx(c + 1)
        start_gather(c % 2)
        if c > 0:
            acc = accum((c - 1) % 2, acc)
    wait_gather((NCHUNK - 1) % 2)
    acc = accum((NCHUNK - 1) % 2, acc)

    @pl.when(wid < NXB)
    def _():
        pltpu.async_copy(idx3_hbm.at[pl.ds(W * BW + wid, 1)],
                         idx3_v.at[pl.ds(0, 1)], isem)
        pltpu.make_async_copy(idx3_hbm.at[pl.ds(0, 1)],
                              idx3_v.at[pl.ds(0, 1)], isem).wait()
        for k in range(HB // L):
            idx1d[pl.ds(k * L, L)] = idx3_v[0, 0, pl.ds(k * L, L)]
            idx1d[pl.ds(HB + k * L, L)] = idx3_v[0, 1, pl.ds(k * L, L)]
        pltpu.async_copy(table_sh.at[idx1d.at[pl.ds(0, 2 * HB)]],
                         val_v0.at[pl.ds(0, 2 * HB)], gsems[0]).wait()
        a2 = acc
        for k in range(HB // L):
            x0 = val_v0[pl.ds(k * L, L)]
            x1 = val_v0[pl.ds(HB + k * L, L)]
            a2 = a2 + jnp.maximum(MARGIN - x0 + x1, 0.0)
        acc_v[...] = a2 * INV_P

    @pl.when(wid >= NXB)
    def _():
        acc_v[...] = acc * INV_P

    pltpu.sync_copy(acc_v, out_hbm.at[wid])


def kernel(outputs, mask):
    idx3 = (
        mask.astype(jnp.int32).T
        .reshape(2, NB, HB)
        .transpose(1, 0, 2)
    )
    parts = _sc_loss(idx3, outputs)
    return jnp.sum(parts)


# speedup vs baseline: 399.1551x; 1.1329x over previous
import functools

import jax
import jax.numpy as jnp
from jax import lax
from jax.experimental import pallas as pl
from jax.experimental.pallas import tpu as pltpu
from jax.experimental.pallas import tpu_sc as plsc

NC = 2
NS = 16
L = 16
W = NC * NS

P = 2_000_000
HB = 128
NB = P // HB
BW = 488
CB = 61
NCHUNK = BW // CB
NXB = NB - W * BW
CPW = CB * 2 * HB
V = 1_000_000
VSTG = 62496
NSP = VSTG // CPW
SREM = VSTG - NSP * CPW
MARGIN = 1.0
INV_P = 1.0 / P


@functools.partial(
    pl.kernel,
    out_type=jax.ShapeDtypeStruct((W, L), jnp.float32),
    mesh=plsc.VectorSubcoreMesh(
        core_axis_name="c", subcore_axis_name="s",
        num_cores=NC, num_subcores=NS),
    scratch_types=[
        pltpu.VMEM((CB, 2, HB), jnp.int32),
        pltpu.VMEM((CPW,), jnp.int32),
        pltpu.VMEM((CPW,), jnp.float32),
        pltpu.VMEM((CPW,), jnp.float32),
        pltpu.VMEM((L,), jnp.float32),
        pltpu.VMEM_SHARED((V,), jnp.float32),
        pltpu.SemaphoreType.DMA,
        [pltpu.SemaphoreType.DMA] * 2,
        pltpu.SemaphoreType.DMA,
    ],
)
def _sc_loss(idx3_hbm, outputs_hbm, out_hbm,
             idx3_v, idx1d, val_v0, val_v1, acc_v, table_sh,
             isem, gsems, ssem):
    sid = lax.axis_index("s")
    wid = sid * NC + lax.axis_index("c")
    base = wid * BW
    val = [val_v0, val_v1]

    def start_idx(c):
        pltpu.async_copy(idx3_hbm.at[pl.ds(base + c * CB, CB)], idx3_v, isem)

    def wait_idx():
        pltpu.make_async_copy(idx3_hbm.at[pl.ds(0, CB)], idx3_v, isem).wait()

    def deinterleave():
        def row(t, _):
            b = t * (2 * HB)
            for k in range(HB // L):
                idx1d[pl.ds(b + k * L, L)] = idx3_v[t, 0, pl.ds(k * L, L)]
                idx1d[pl.ds(b + HB + k * L, L)] = idx3_v[t, 1, pl.ds(k * L, L)]
            return 0
        lax.fori_loop(0, CB, row, 0)

    def start_gather(b):
        pltpu.async_copy(table_sh.at[idx1d], val[b], gsems[b])

    def wait_gather(b):
        pltpu.make_async_copy(table_sh.at[idx1d], val[b], gsems[b]).wait()

    def accum(b, acc):
        def it(t, a):
            bb = t * (2 * HB)
            for k in range(HB // L):
                x0 = val[b][pl.ds(bb + k * L, L)]
                x1 = val[b][pl.ds(bb + HB + k * L, L)]
                a = a + jnp.maximum(MARGIN - x0 + x1, 0.0)
            return a
        return lax.fori_loop(0, CB, it, acc)

    start_idx(0)

    for p in range(NSP):
        b = p % 2
        poff = pl.multiple_of(sid * VSTG + p * CPW, 8)
        if p >= 2:
            pltpu.make_async_copy(val[b], table_sh.at[pl.ds(0, CPW)],
                                  ssem).wait()
        pltpu.async_copy(outputs_hbm.at[pl.ds(poff, CPW)], val[b],
                         gsems[b]).wait()
        pltpu.async_copy(val[b], table_sh.at[pl.ds(poff, CPW)], ssem)
    pltpu.make_async_copy(val[0], table_sh.at[pl.ds(0, CPW)], ssem).wait()
    pltpu.make_async_copy(val[1], table_sh.at[pl.ds(0, CPW)], ssem).wait()

    roff = pl.multiple_of(sid * VSTG + NSP * CPW, 8)
    pltpu.async_copy(outputs_hbm.at[pl.ds(roff, SREM)],
                     val_v0.at[pl.ds(0, SREM)], ssem).wait()
    pltpu.async_copy(val_v0.at[pl.ds(0, SREM)],
                     table_sh.at[pl.ds(roff, SREM)], ssem).wait()

    @pl.when(sid == NS - 1)
    def _():
        poff = pl.multiple_of(NS * VSTG, 8)
        rem = V - NS * VSTG
        pltpu.async_copy(outputs_hbm.at[pl.ds(poff, rem)],
                         val_v1.at[pl.ds(0, rem)], ssem).wait()
        pltpu.async_copy(val_v1.at[pl.ds(0, rem)],
                         table_sh.at[pl.ds(poff, rem)], ssem).wait()

    plsc.subcore_barrier()

    acc = jnp.zeros((L,), jnp.float32)
    for c in range(NCHUNK):
        wait_idx()
        if c > 0:
            wait_gather((c - 1) % 2)
        deinterleave()
        if c < NCHUNK - 1:
            start_idx(c + 1)
        start_gather(c % 2)
        if c > 0:
            acc = accum((c - 1) % 2, acc)
    wait_gather((NCHUNK - 1) % 2)
    acc = accum((NCHUNK - 1) % 2, acc)

    @pl.when(wid < NXB)
    def _():
        pltpu.async_copy(idx3_hbm.at[pl.ds(W * BW + wid, 1)],
                         idx3_v.at[pl.ds(0, 1)], isem)
        pltpu.make_async_copy(idx3_hbm.at[pl.ds(0, 1)],
                              idx3_v.at[pl.ds(0, 1)], isem).wait()
        for k in range(HB // L):
            idx1d[pl.ds(k * L, L)] = idx3_v[0, 0, pl.ds(k * L, L)]
            idx1d[pl.ds(HB + k * L, L)] = idx3_v[0, 1, pl.ds(k * L, L)]
        pltpu.async_copy(table_sh.at[idx1d.at[pl.ds(0, 2 * HB)]],
                         val_v0.at[pl.ds(0, 2 * HB)], gsems[0]).wait()
        a2 = acc
        for k in range(HB // L):
            x0 = val_v0[pl.ds(k * L, L)]
            x1 = val_v0[pl.ds(HB + k * L, L)]
            a2 = a2 + jnp.maximum(MARGIN - x0 + x1, 0.0)
        acc_v[...] = a2 * INV_P

    @pl.when(wid >= NXB)
    def _():
        acc_v[...] = acc * INV_P

    pltpu.sync_copy(acc_v, out_hbm.at[wid])


def kernel(outputs, mask):
    idx3 = (
        mask.astype(jnp.int32).T
        .reshape(2, NB, HB)
        .transpose(1, 0, 2)
    )
    parts = _sc_loss(idx3, outputs)
    return jnp.sum(parts)
